# Initial kernel scaffold; baseline (speedup 1.0000x reference)
import jax
import jax.numpy as jnp
from jax.experimental import pallas as pl


def kernel(x, index_scores):
    raise NotImplementedError("write your pallas kernel here")


# speedup vs baseline: 5.9636x; 5.9636x over previous
import functools

import jax
import jax.numpy as jnp
import numpy as np
from jax import lax
from jax.experimental import pallas as pl
from jax.experimental.pallas import tpu as pltpu
from jax.experimental.pallas import tpu_sc as plsc

_RATIO = 0.5
_MIN_K = 16
_MAX_K = 512

_NUM_CORES = 2
_NUM_SUBCORES = 16
_NUM_WORKERS = _NUM_CORES * _NUM_SUBCORES
_L = 16

_INT_MIN = np.int32(-2147483648)


def _worker_id():
    return lax.axis_index("s") * _NUM_CORES + lax.axis_index("c")


def _f32_to_key(v):
    i = plsc.bitcast(v, jnp.int32)
    flip = lax.shift_right_arithmetic(i, jnp.int32(31)) & jnp.int32(0x7FFFFFFF)
    return i ^ flip


def _selector_body(k, S, n_rows, scores_hbm, out_hbm, row_f, key_v, mask_v):
    rows_per_worker = n_rows // _NUM_WORKERS
    nvec = S // _L
    wid = _worker_id()
    lane = lax.iota(jnp.int32, _L)

    def do_row(t, _):
        r = wid + t * _NUM_WORKERS
        q = lax.rem(r, S)
        heavy = q >= k

        @pl.when(heavy)
        def _heavy():
            pltpu.sync_copy(scores_hbm.at[r], row_f)
            nv = q // _L + 1

            def conv(j, _):
                v = row_f[pl.ds(j * _L, _L)]
                key = _f32_to_key(v)
                col = j * _L + lane
                key_v[pl.ds(j * _L, _L)] = jnp.where(col <= q, key, _INT_MIN)
                return 0

            lax.fori_loop(0, nv, conv, 0)

            def bit_step(b, U):
                Ucand = U | lax.shift_left(jnp.int32(1), jnp.int32(31) - b)
                T = Ucand ^ _INT_MIN
                Tv = jnp.broadcast_to(T, (_L,))

                def cnt_step(j, acc):
                    kv = key_v[pl.ds(j * _L, _L)]
                    return acc + jnp.where(kv >= Tv, jnp.int32(1), jnp.int32(0))

                acc = lax.fori_loop(0, nv, cnt_step, jnp.zeros((_L,), jnp.int32))
                cnt = jnp.sum(acc)
                return jnp.where(cnt >= k, Ucand, U)

            U = lax.fori_loop(0, 32, bit_step, jnp.int32(0))
            Tv = jnp.broadcast_to(U ^ _INT_MIN, (_L,))

            def pad(j, _):
                key_v[pl.ds(j * _L, _L)] = jnp.broadcast_to(_INT_MIN, (_L,))
                return 0

            lax.fori_loop(nv, nvec, pad, 0)

            def emit(j, _):
                kv = key_v[pl.ds(j * _L, _L)]
                mask_v[pl.ds(j * _L, _L)] = jnp.where(
                    kv >= Tv, jnp.int32(1), jnp.int32(0))
                return 0

            lax.fori_loop(0, nvec, emit, 0)
            pltpu.sync_copy(mask_v, out_hbm.at[r])

        @pl.when(jnp.logical_not(heavy))
        def _light():
            def emit(j, _):
                col = j * _L + lane
                mask_v[pl.ds(j * _L, _L)] = jnp.where(
                    col <= q, jnp.int32(1), jnp.int32(0))
                return 0

            lax.fori_loop(0, nvec, emit, 0)
            pltpu.sync_copy(mask_v, out_hbm.at[r])

        return 0

    lax.fori_loop(0, rows_per_worker, do_row, 0)


def kernel(x, index_scores):
    B, S, _ = x.shape
    k = int(S * _RATIO)
    k = max(_MIN_K, min(_MAX_K, k))

    scores = index_scores.reshape(B * S, S)
    mesh = plsc.VectorSubcoreMesh(
        core_axis_name="c", subcore_axis_name="s",
        num_cores=_NUM_CORES, num_subcores=_NUM_SUBCORES)

    body = functools.partial(_selector_body, k, S, B * S)
    mask_i32 = pl.kernel(
        body,
        out_type=jax.ShapeDtypeStruct((B * S, S), jnp.int32),
        mesh=mesh,
        scratch_types=[
            pltpu.VMEM((S,), jnp.float32),
            pltpu.VMEM((S,), jnp.int32),
            pltpu.VMEM((S,), jnp.int32),
        ],
        compiler_params=pltpu.CompilerParams(needs_layout_passes=False),
    )(scores)

    top_k_mask = mask_i32.astype(bool).reshape(B, S, S)
    k_values = jnp.full((B, S), k, dtype=jnp.int32)
    return (top_k_mask, k_values)
